# baseline (speedup 1.0000x reference)
import functools

import jax
import jax.numpy as jnp
from jax import lax
from jax.experimental import pallas as pl
from jax.experimental.pallas import tpu as pltpu
from jax.experimental.pallas import tpu_sc as plsc

B, V, D = 4, 10000, 16
N = B * V * D
BV = B * V
TROWS = B * (V + 1)
VE = EE = CE = 16
T = 2
F32 = jnp.float32

NW = 32
PER_W = N // NW
CH = 400
NCH = PER_W // CH


def _sc_gather(tables, idxs):
    ns = len(tables)
    mesh = plsc.VectorSubcoreMesh(core_axis_name="c", subcore_axis_name="s")
    scratch = ([pltpu.VMEM((PER_W,), jnp.int32)] * ns
               + [pltpu.VMEM((CH, 16), F32)] * (2 * ns)
               + [pltpu.SemaphoreType.DMA] * (2 * ns))

    @functools.partial(
        pl.kernel,
        mesh=mesh,
        compiler_params=pltpu.CompilerParams(use_tc_tiling_on_sc=False),
        out_type=tuple(jax.ShapeDtypeStruct((N, 16), F32) for _ in range(ns)),
        scratch_types=scratch,
    )
    def gk(*refs):
        t_hbm = refs[0:ns]
        i_hbm = refs[ns:2 * ns]
        o_hbm = refs[2 * ns:3 * ns]
        k = 3 * ns
        i_v = refs[k:k + ns]
        bufs = [(refs[k + ns + 2 * j], refs[k + ns + 2 * j + 1]) for j in range(ns)]
        sems = [(refs[k + 3 * ns + 2 * j], refs[k + 3 * ns + 2 * j + 1])
                for j in range(ns)]
        wid = lax.axis_index("s") * 2 + lax.axis_index("c")
        base = wid * PER_W

        for j in range(ns):
            pltpu.sync_copy(i_hbm[j].at[pl.ds(base, PER_W)], i_v[j])

        def dsc(j, c, s):
            return pltpu.make_async_copy(
                t_hbm[j].at[i_v[j].at[pl.ds(c * CH, CH)]], bufs[j][s], sems[j][s])

        def start(c, s):
            for j in range(ns):
                dsc(j, c, s).start()

        def wait(c, s):
            for j in range(ns):
                dsc(j, c, s).wait()

        for s in range(2):
            start(s, s)

        def body(i, carry):
            for s in range(2):
                c = 2 * i + s
                wait(c, s)
                for j in range(ns):
                    pltpu.sync_copy(bufs[j][s],
                                    o_hbm[j].at[pl.ds(base + c * CH, CH)])

                @pl.when(c + 2 < NCH)
                def _():
                    start(c + 2, s)
            return carry

        lax.fori_loop(0, NCH // 2, body, 0)

    return gk(*tables, *idxs)


VB = 400
EB = VB * D
GRID = BV // VB
GPB = V // VB


def _relu(x):
    return jnp.maximum(x, 0.0)


def _full(w):
    return pl.BlockSpec(w.shape, lambda *g: tuple(0 for _ in w.shape))


def _enc_v_body(vx, ctx, wv1, bv1, wv2, bv2, wv3, bv3,
                wc1, bc1, wc2, bc2, wc3, bc3, out_v, out_c):
    h = _relu(vx[...] @ wv1[...] + bv1[...])
    h = _relu(h @ wv2[...] + bv2[...])
    out_v[...] = h @ wv3[...] + bv3[...]

    @pl.when(pl.program_id(0) == 0)
    def _():
        hc = _relu(ctx[...] @ wc1[...] + bc1[...])
        hc = _relu(hc @ wc2[...] + bc2[...])
        out_c[...] = hc @ wc3[...] + bc3[...]


def _enc_v(vertex_f, context, pv, pc):
    vb = 2000
    ws = (pv['W1'], pv['b1'], pv['W2'], pv['b2'], pv['W3'], pv['b3'],
          pc['W1'], pc['b1'], pc['W2'], pc['b2'], pc['W3'], pc['b3'])
    return pl.pallas_call(
        _enc_v_body,
        grid=(BV // vb,),
        in_specs=[pl.BlockSpec((vb, 8), lambda g: (g, 0)),
                  _full(context)] + [_full(w) for w in ws],
        out_specs=[pl.BlockSpec((vb, VE), lambda g: (g, 0)),
                   pl.BlockSpec((B, CE), lambda g: (0, 0))],
        out_shape=[jax.ShapeDtypeStruct((BV, VE), F32),
                   jax.ShapeDtypeStruct((B, CE), F32)],
    )(vertex_f, context, *ws)


def _mega_tail(mask16, inc, embv, embc, h, wv, o_e, o_oe, o_v, o_ov, o_ge, o_gv, o_ne):
    g = pl.program_id(0)
    b = g // GPB
    h = _relu(h @ wv['w2'][...] + wv['b2'][...])
    o80 = h @ wv['w3d'][...] + wv['bd'][...]
    e_new = o80[:, 0:16]
    o_e[...] = e_new
    dh = _relu(o80[:, 16:80])
    dh = _relu(dh @ wv['d2'][...] + wv['dd2'][...])
    w = dh @ wv['d3q'][...] + wv['dd3q'][...]
    sp = jnp.maximum(w, 0.0) + jnp.log1p(jnp.exp(-jnp.abs(w)))
    o_oe[...] = jnp.where(mask16[:, 0:2] > 0.0, -sp, 0.0)

    m_e = mask16 * e_new
    es = jnp.sum(m_e.reshape(VB, D, EE), axis=1)
    ns = jnp.sum((inc[...] > 0).astype(F32), axis=1, keepdims=True)
    avg_e = jnp.where(ns > 0.0, es / jnp.maximum(ns, 1.0), 0.0)
    xv = jnp.concatenate([avg_e, embv[...]], axis=1)
    ccv = embc[pl.ds(b, 1), :] @ wv['wv1c'][...]
    hv = _relu(xv @ wv['wv1'][...] + ccv + wv['bv1'][...])
    hv = _relu(hv @ wv['wv2'][...] + wv['bv2'][...])
    ov80 = hv @ wv['wv3d'][...] + wv['bvd'][...]
    v_new = ov80[:, 0:16]
    o_v[...] = v_new
    dv = _relu(ov80[:, 16:80])
    dv = _relu(dv @ wv['dv2'][...] + wv['ddv2'][...])
    o_ov[...] = dv @ wv['dv3'][...] + wv['ddv3'][...]

    @pl.when(g == 0)
    def _():
        o_ge[...] = jnp.zeros_like(o_ge)
        o_gv[...] = jnp.zeros_like(o_gv)
        o_ne[...] = jnp.zeros_like(o_ne)

    ones1 = jnp.ones((1, VB), F32)
    o_ge[pl.ds(b, 1), :] += ones1 @ es
    o_gv[pl.ds(b, 1), :] += ones1 @ v_new
    o_ne[pl.ds(b, 1), :] += jnp.broadcast_to(ones1 @ ns, (1, 16))


_TAIL_KEYS = ('w2', 'b2', 'w3d', 'bd', 'd2', 'dd2', 'd3q', 'dd3q',
              'wv1', 'wv1c', 'bv1', 'wv2', 'bv2', 'wv3d', 'bvd',
              'dv2', 'ddv2', 'dv3', 'ddv3')
_T0_KEYS = ('we1', 'be1', 'we2', 'be2', 'wfa', 'wfb', 'w1c', 'b1e') + _TAIL_KEYS
_T1_KEYS = ('wfa', 'w1c', 'b1e') + _TAIL_KEYS


def _mega_body_t0(ein, sv, rv, mk, inc, embv, embc, *rest):
    nw = len(_T0_KEYS)
    wv = dict(zip(_T0_KEYS, rest[:nw]))
    outs = rest[nw:]
    mask16 = mk[...]
    he = _relu(ein[...] @ wv['we1'][...] + wv['be1'][...])
    he = _relu(he @ wv['we2'][...] + wv['be2'][...])
    x = jnp.concatenate([mask16 * rv[...], mask16 * sv[...]], axis=1)
    cc = embc[pl.ds(pl.program_id(0) // GPB, 1), :] @ wv['w1c'][...]
    h = _relu(he @ wv['wfa'][...] + x @ wv['wfb'][...] + cc + wv['b1e'][...])
    _mega_tail(mask16, inc, embv, embc, h, wv, *outs)


def _mega_body_t1(ein, sv, rv, mk, inc, embv, embc, *rest):
    nw = len(_T1_KEYS)
    wv = dict(zip(_T1_KEYS, rest[:nw]))
    outs = rest[nw:]
    mask16 = mk[...]
    x = jnp.concatenate([ein[...], mask16 * rv[...], mask16 * sv[...]], axis=1)
    cc = embc[pl.ds(pl.program_id(0) // GPB, 1), :] @ wv['w1c'][...]
    h = _relu(x @ wv['wfa'][...] + cc + wv['b1e'][...])
    _mega_tail(mask16, inc, embv, embc, h, wv, *outs)


def _mega(t0, ein, sv, rv, mk, inc2d, embv, embc, wd):
    keys = _T0_KEYS if t0 else _T1_KEYS
    ein_w = ein.shape[1]
    body = _mega_body_t0 if t0 else _mega_body_t1
    return pl.pallas_call(
        body,
        grid=(GRID,),
        in_specs=[
            pl.BlockSpec((EB, ein_w), lambda g: (g, 0)),
            pl.BlockSpec((EB, 16), lambda g: (g, 0)),
            pl.BlockSpec((EB, 16), lambda g: (g, 0)),
            pl.BlockSpec((EB, 16), lambda g: (g, 0)),
            pl.BlockSpec((VB, D), lambda g: (g, 0)),
            pl.BlockSpec((VB, VE), lambda g: (g, 0)),
            pl.BlockSpec((B, CE), lambda g: (0, 0)),
        ] + [_full(wd[k]) for k in keys],
        out_specs=[
            pl.BlockSpec((EB, EE), lambda g: (g, 0)),
            pl.BlockSpec((EB, 2), lambda g: (g, 0)),
            pl.BlockSpec((VB, VE), lambda g: (g, 0)),
            pl.BlockSpec((VB, 2), lambda g: (g, 0)),
            pl.BlockSpec((B, EE), lambda g: (0, 0)),
            pl.BlockSpec((B, VE), lambda g: (0, 0)),
            pl.BlockSpec((B, 16), lambda g: (0, 0)),
        ],
        out_shape=[
            jax.ShapeDtypeStruct((N, EE), F32),
            jax.ShapeDtypeStruct((N, 2), F32),
            jax.ShapeDtypeStruct((BV, VE), F32),
            jax.ShapeDtypeStruct((BV, 2), F32),
            jax.ShapeDtypeStruct((B, EE), F32),
            jax.ShapeDtypeStruct((B, VE), F32),
            jax.ShapeDtypeStruct((B, 16), F32),
        ],
    )(ein, sv, rv, mk, inc2d, embv, embc, *[wd[k] for k in keys])


def _ctx_body(ge, gv, ne, embc, w1e, w1v, w1c, b1, w2, b2, w3d, bd,
              dc2, ddc2, dc3, ddc3, o_c, o_oc):
    gee = ge[...] / ne[:, 0:1]
    gvv = gv[...] * (1.0 / V)
    h = _relu(gee @ w1e[...] + gvv @ w1v[...] + embc[...] @ w1c[...] + b1[...])
    h = _relu(h @ w2[...] + b2[...])
    o80 = h @ w3d[...] + bd[...]
    o_c[...] = o80[:, 0:16]
    dh = _relu(o80[:, 16:80])
    dh = _relu(dh @ dc2[...] + ddc2[...])
    o_oc[...] = dh @ dc3[...] + ddc3[...]


def _ctx(ge, gv, ne, embc, cw):
    args = (ge, gv, ne, embc) + cw
    return pl.pallas_call(
        _ctx_body,
        in_specs=[_full(a) for a in args],
        out_specs=[pl.BlockSpec((B, CE), lambda *g: (0, 0)),
                   pl.BlockSpec((B, 2), lambda *g: (0, 0))],
        out_shape=[jax.ShapeDtypeStruct((B, CE), F32),
                   jax.ShapeDtypeStruct((B, 2), F32)],
    )(*args)


def _prep_weights(params):
    p_e, p_v, p_c = params['e_net'], params['v_net'], params['c_net']
    d_e, d_v, d_c = params['dec_e'], params['dec_v'], params['dec_c']
    enc_e = params['enc_e']

    def row(b):
        return b.reshape(1, -1)

    wd = {}
    w1 = p_e['W1']
    wd['w1c'] = w1[48:64]
    wd['b1e'] = row(p_e['b1'])
    wd['w2'] = p_e['W2']
    wd['b2'] = row(p_e['b2'])
    wd['w3d'] = jnp.concatenate([p_e['W3'], p_e['W3'] @ d_e['W1']], axis=1)
    wd['bd'] = row(jnp.concatenate([p_e['b3'], p_e['b3'] @ d_e['W1'] + d_e['b1']]))
    wd['d2'] = d_e['W2']
    wd['dd2'] = row(d_e['b2'])
    q = jnp.array([[-1.0, 1.0], [1.0, -1.0]], F32)
    wd['d3q'] = d_e['W3'] @ q
    wd['dd3q'] = row(d_e['b3'] @ q)
    wv1 = p_v['W1']
    wd['wv1'] = wv1[0:32]
    wd['wv1c'] = wv1[32:48]
    wd['bv1'] = row(p_v['b1'])
    wd['wv2'] = p_v['W2']
    wd['bv2'] = row(p_v['b2'])
    wd['wv3d'] = jnp.concatenate([p_v['W3'], p_v['W3'] @ d_v['W1']], axis=1)
    wd['bvd'] = row(jnp.concatenate([p_v['b3'], p_v['b3'] @ d_v['W1'] + d_v['b1']]))
    wd['dv2'] = d_v['W2']
    wd['ddv2'] = row(d_v['b2'])
    wd['dv3'] = d_v['W3']
    wd['ddv3'] = row(d_v['b3'])
    wd['we1'] = enc_e['W1']
    wd['be1'] = row(enc_e['b1'])
    wd['we2'] = enc_e['W2']
    wd['be2'] = row(enc_e['b2'])
    wd['wfa0'] = enc_e['W3'] @ w1[0:16]
    wd['wfb0'] = w1[16:48]
    wd['b1e0'] = row(p_e['b1'] + enc_e['b3'] @ w1[0:16])
    wd['wfa1'] = w1[0:48]
    wc1 = p_c['W1']
    cw = (wc1[0:16], wc1[16:32], wc1[32:48], row(p_c['b1']),
          p_c['W2'], row(p_c['b2']),
          jnp.concatenate([p_c['W3'], p_c['W3'] @ d_c['W1']], axis=1),
          row(jnp.concatenate([p_c['b3'], p_c['b3'] @ d_c['W1'] + d_c['b1']])),
          d_c['W2'], row(d_c['b2']), d_c['W3'], row(d_c['b3']))
    return wd, cw


def kernel(vertex, edge, context, incoming, params):
    wd, cw = _prep_weights(params)
    vertex_f = vertex.reshape(BV, 8)
    edge_f = edge.reshape(N, 4)
    inc2d = incoming.reshape(BV, D)

    boff = (jnp.arange(B, dtype=jnp.int32) * V)[:, None, None]
    idx_s = (jnp.maximum(incoming - 1, 0) + boff).reshape(N)
    vrow = jnp.arange(V, dtype=jnp.int32)[None, :, None]
    idx_r = jnp.broadcast_to(vrow + boff, (B, V, D)).reshape(N)
    idx_m = (incoming > 0).astype(jnp.int32).reshape(N)
    mtab = jnp.concatenate(
        [jnp.zeros((1, 16), F32), jnp.ones((1, 16), F32)], axis=0)

    emb_v, emb_c = _enc_v(vertex_f, context, params['enc_v'], params['enc_c'])

    mk = None
    outs_v, outs_e, outs_c = [], [], []
    ein = edge_f
    for t in range(T):
        if mk is None:
            sv, rv, mk = _sc_gather((emb_v, emb_v, mtab), (idx_s, idx_r, idx_m))
        else:
            sv, rv = _sc_gather((emb_v, emb_v), (idx_s, idx_r))

        wd_t = dict(wd)
        if t == 0:
            wd_t['wfa'], wd_t['wfb'], wd_t['b1e'] = wd['wfa0'], wd['wfb0'], wd['b1e0']
        else:
            wd_t['wfa'] = wd['wfa1']
        e_new, o_e, v_new, o_v, ge, gv, ne = _mega(
            t == 0, ein, sv, rv, mk, inc2d, emb_v, emb_c, wd_t)
        emb_c, o_c = _ctx(ge, gv, ne, emb_c, cw)
        emb_v = v_new
        ein = e_new
        outs_v.append(o_v.reshape(B, V, 2))
        outs_e.append(o_e.reshape(B, V, D, 2))
        outs_c.append(o_c)

    return (jnp.stack(outs_v), jnp.stack(outs_e), jnp.stack(outs_c))


# speedup vs baseline: 1.0524x; 1.0524x over previous
import functools

import jax
import jax.numpy as jnp
from jax import lax
from jax.experimental import pallas as pl
from jax.experimental.pallas import tpu as pltpu
from jax.experimental.pallas import tpu_sc as plsc

B, V, D = 4, 10000, 16
N = B * V * D
BV = B * V
TROWS = B * (V + 1)
VE = EE = CE = 16
T = 2
F32 = jnp.float32

NW = 32
PER_W = N // NW
CH = 400
NCH = PER_W // CH


def _sc_gather(tables, idxs):
    ns = len(tables)
    mesh = plsc.VectorSubcoreMesh(core_axis_name="c", subcore_axis_name="s")
    scratch = ([pltpu.VMEM((PER_W,), jnp.int32)] * ns
               + [pltpu.VMEM((CH, 16), F32)] * (2 * ns)
               + [pltpu.SemaphoreType.DMA] * (2 * ns))

    @functools.partial(
        pl.kernel,
        mesh=mesh,
        compiler_params=pltpu.CompilerParams(use_tc_tiling_on_sc=False),
        out_type=tuple(jax.ShapeDtypeStruct((N, 16), F32) for _ in range(ns)),
        scratch_types=scratch,
    )
    def gk(*refs):
        t_hbm = refs[0:ns]
        i_hbm = refs[ns:2 * ns]
        o_hbm = refs[2 * ns:3 * ns]
        k = 3 * ns
        i_v = refs[k:k + ns]
        bufs = [(refs[k + ns + 2 * j], refs[k + ns + 2 * j + 1]) for j in range(ns)]
        sems = [(refs[k + 3 * ns + 2 * j], refs[k + 3 * ns + 2 * j + 1])
                for j in range(ns)]
        wid = lax.axis_index("s") * 2 + lax.axis_index("c")
        base = wid * PER_W

        for j in range(ns):
            pltpu.sync_copy(i_hbm[j].at[pl.ds(base, PER_W)], i_v[j])

        def dsc(j, c, s):
            return pltpu.make_async_copy(
                t_hbm[j].at[i_v[j].at[pl.ds(c * CH, CH)]], bufs[j][s], sems[j][s])

        def start(c, s):
            for j in range(ns):
                dsc(j, c, s).start()

        def wait(c, s):
            for j in range(ns):
                dsc(j, c, s).wait()

        for s in range(2):
            start(s, s)

        def body(i, carry):
            for s in range(2):
                c = 2 * i + s
                wait(c, s)
                for j in range(ns):
                    pltpu.sync_copy(bufs[j][s],
                                    o_hbm[j].at[pl.ds(base + c * CH, CH)])

                @pl.when(c + 2 < NCH)
                def _():
                    start(c + 2, s)
            return carry

        lax.fori_loop(0, NCH // 2, body, 0)

    return gk(*tables, *idxs)


VB = 400
EB = VB * D
GRID = BV // VB
GPB = V // VB
PK = 4
EBP = EB // PK
NP = N // PK


def _relu(x):
    return jnp.maximum(x, 0.0)


def _full(w):
    return pl.BlockSpec(w.shape, lambda *g: tuple(0 for _ in w.shape))


def _enc_v_body(vx, ctx, wv1, bv1, wv2, bv2, wv3, bv3,
                wc1, bc1, wc2, bc2, wc3, bc3, out_v, out_c):
    h = _relu(vx[...] @ wv1[...] + bv1[...])
    h = _relu(h @ wv2[...] + bv2[...])
    out_v[...] = h @ wv3[...] + bv3[...]

    @pl.when(pl.program_id(0) == 0)
    def _():
        hc = _relu(ctx[...] @ wc1[...] + bc1[...])
        hc = _relu(hc @ wc2[...] + bc2[...])
        out_c[...] = hc @ wc3[...] + bc3[...]


def _enc_v(vertex_f, context, pv, pc):
    vb = 2000
    ws = (pv['W1'], pv['b1'], pv['W2'], pv['b2'], pv['W3'], pv['b3'],
          pc['W1'], pc['b1'], pc['W2'], pc['b2'], pc['W3'], pc['b3'])
    return pl.pallas_call(
        _enc_v_body,
        grid=(BV // vb,),
        in_specs=[pl.BlockSpec((vb, 8), lambda g: (g, 0)),
                  _full(context)] + [_full(w) for w in ws],
        out_specs=[pl.BlockSpec((vb, VE), lambda g: (g, 0)),
                   pl.BlockSpec((B, CE), lambda g: (0, 0))],
        out_shape=[jax.ShapeDtypeStruct((BV, VE), F32),
                   jax.ShapeDtypeStruct((B, CE), F32)],
    )(vertex_f, context, *ws)


def _mega_tail(mask16, inc, embv, embc, h, wv, o_e, o_oe, o_v, o_ov, o_ge, o_gv, o_ne):
    g = pl.program_id(0)
    b = g // GPB
    h = _relu(h @ wv['w2'][...] + wv['b2'][...])
    e_new = h @ wv['w3'][...] + wv['b3'][...]
    o_e[...] = e_new
    dh = _relu(h @ wv['wd1'][...] + wv['bd1'][...])
    dh = _relu(dh @ wv['d2'][...] + wv['dd2'][...])
    w = dh @ wv['d3q'][...] + wv['dd3q'][...]
    sp = jnp.maximum(w, 0.0) + jnp.log(1.0 + jnp.exp(-jnp.abs(w)))
    mask2 = jnp.concatenate([mask16[:, 0:2], mask16[:, 16:18],
                             mask16[:, 32:34], mask16[:, 48:50]], axis=1)
    o_oe[...] = jnp.where(mask2 > 0.0, -sp, 0.0)

    m_e = mask16 * e_new
    es4 = jnp.sum(m_e.reshape(VB, PK, 64), axis=1)
    es = (es4[:, 0:16] + es4[:, 16:32] + es4[:, 32:48] + es4[:, 48:64])
    ns = jnp.sum((inc[...] > 0).astype(F32), axis=1, keepdims=True)
    avg_e = jnp.where(ns > 0.0, es / jnp.maximum(ns, 1.0), 0.0)
    xv = jnp.concatenate([avg_e, embv[...]], axis=1)
    ccv = embc[pl.ds(b, 1), :] @ wv['wv1c'][...]
    hv = _relu(xv @ wv['wv1'][...] + ccv + wv['bv1'][...])
    hv = _relu(hv @ wv['wv2'][...] + wv['bv2'][...])
    ov80 = hv @ wv['wv3d'][...] + wv['bvd'][...]
    v_new = ov80[:, 0:16]
    o_v[...] = v_new
    dv = _relu(ov80[:, 16:80])
    dv = _relu(dv @ wv['dv2'][...] + wv['ddv2'][...])
    o_ov[...] = dv @ wv['dv3'][...] + wv['ddv3'][...]

    @pl.when(g == 0)
    def _():
        o_ge[...] = jnp.zeros_like(o_ge)
        o_gv[...] = jnp.zeros_like(o_gv)
        o_ne[...] = jnp.zeros_like(o_ne)

    ones1 = jnp.ones((1, VB), F32)
    o_ge[pl.ds(b, 1), :] += ones1 @ es
    o_gv[pl.ds(b, 1), :] += ones1 @ v_new
    o_ne[pl.ds(b, 1), :] += jnp.broadcast_to(ones1 @ ns, (1, 16))


_TAIL_KEYS = ('w2', 'b2', 'w3', 'b3', 'wd1', 'bd1', 'd2', 'dd2', 'd3q', 'dd3q',
              'wv1', 'wv1c', 'bv1', 'wv2', 'bv2', 'wv3d', 'bvd',
              'dv2', 'ddv2', 'dv3', 'ddv3')
_T0_KEYS = ('we1', 'be1', 'we2', 'be2', 'wfa', 'wfb', 'w1c', 'b1e') + _TAIL_KEYS
_T1_KEYS = ('wfa', 'w1c', 'b1e') + _TAIL_KEYS


def _mega_body_t0(ein, sv, rv, mk, inc, embv, embc, *rest):
    nw = len(_T0_KEYS)
    wv = dict(zip(_T0_KEYS, rest[:nw]))
    outs = rest[nw:]
    mask16 = mk[...]
    he = _relu(ein[...] @ wv['we1'][...] + wv['be1'][...])
    he = _relu(he @ wv['we2'][...] + wv['be2'][...])
    x = jnp.concatenate([mask16 * rv[...], mask16 * sv[...]], axis=1)
    cc = embc[pl.ds(pl.program_id(0) // GPB, 1), :] @ wv['w1c'][...]
    h = _relu(he @ wv['wfa'][...] + x @ wv['wfb'][...] + cc + wv['b1e'][...])
    _mega_tail(mask16, inc, embv, embc, h, wv, *outs)


def _mega_body_t1(ein, sv, rv, mk, inc, embv, embc, *rest):
    nw = len(_T1_KEYS)
    wv = dict(zip(_T1_KEYS, rest[:nw]))
    outs = rest[nw:]
    mask16 = mk[...]
    x = jnp.concatenate([ein[...], mask16 * rv[...], mask16 * sv[...]], axis=1)
    cc = embc[pl.ds(pl.program_id(0) // GPB, 1), :] @ wv['w1c'][...]
    h = _relu(x @ wv['wfa'][...] + cc + wv['b1e'][...])
    _mega_tail(mask16, inc, embv, embc, h, wv, *outs)


def _mega(t0, ein, sv, rv, mk, inc2d, embv, embc, wd):
    keys = _T0_KEYS if t0 else _T1_KEYS
    ein_w = ein.shape[1]
    body = _mega_body_t0 if t0 else _mega_body_t1
    return pl.pallas_call(
        body,
        grid=(GRID,),
        in_specs=[
            pl.BlockSpec((EBP, ein_w), lambda g: (g, 0)),
            pl.BlockSpec((EBP, 64), lambda g: (g, 0)),
            pl.BlockSpec((EBP, 64), lambda g: (g, 0)),
            pl.BlockSpec((EBP, 64), lambda g: (g, 0)),
            pl.BlockSpec((VB, D), lambda g: (g, 0)),
            pl.BlockSpec((VB, VE), lambda g: (g, 0)),
            pl.BlockSpec((B, CE), lambda g: (0, 0)),
        ] + [_full(wd[k]) for k in keys],
        out_specs=[
            pl.BlockSpec((EBP, 64), lambda g: (g, 0)),
            pl.BlockSpec((EBP, 8), lambda g: (g, 0)),
            pl.BlockSpec((VB, VE), lambda g: (g, 0)),
            pl.BlockSpec((VB, 2), lambda g: (g, 0)),
            pl.BlockSpec((B, EE), lambda g: (0, 0)),
            pl.BlockSpec((B, VE), lambda g: (0, 0)),
            pl.BlockSpec((B, 16), lambda g: (0, 0)),
        ],
        out_shape=[
            jax.ShapeDtypeStruct((NP, 64), F32),
            jax.ShapeDtypeStruct((NP, 8), F32),
            jax.ShapeDtypeStruct((BV, VE), F32),
            jax.ShapeDtypeStruct((BV, 2), F32),
            jax.ShapeDtypeStruct((B, EE), F32),
            jax.ShapeDtypeStruct((B, VE), F32),
            jax.ShapeDtypeStruct((B, 16), F32),
        ],
    )(ein, sv, rv, mk, inc2d, embv, embc, *[wd[k] for k in keys])


def _ctx_body(ge, gv, ne, embc, w1e, w1v, w1c, b1, w2, b2, w3d, bd,
              dc2, ddc2, dc3, ddc3, o_c, o_oc):
    gee = ge[...] / ne[:, 0:1]
    gvv = gv[...] * (1.0 / V)
    h = _relu(gee @ w1e[...] + gvv @ w1v[...] + embc[...] @ w1c[...] + b1[...])
    h = _relu(h @ w2[...] + b2[...])
    o80 = h @ w3d[...] + bd[...]
    o_c[...] = o80[:, 0:16]
    dh = _relu(o80[:, 16:80])
    dh = _relu(dh @ dc2[...] + ddc2[...])
    o_oc[...] = dh @ dc3[...] + ddc3[...]


def _ctx(ge, gv, ne, embc, cw):
    args = (ge, gv, ne, embc) + cw
    return pl.pallas_call(
        _ctx_body,
        in_specs=[_full(a) for a in args],
        out_specs=[pl.BlockSpec((B, CE), lambda *g: (0, 0)),
                   pl.BlockSpec((B, 2), lambda *g: (0, 0))],
        out_shape=[jax.ShapeDtypeStruct((B, CE), F32),
                   jax.ShapeDtypeStruct((B, 2), F32)],
    )(*args)


def _prep_weights(params):
    p_e, p_v, p_c = params['e_net'], params['v_net'], params['c_net']
    d_e, d_v, d_c = params['dec_e'], params['dec_v'], params['dec_c']
    enc_e = params['enc_e']

    def row(b):
        return b.reshape(1, -1)

    eye4 = jnp.eye(PK, dtype=F32)

    def bd4(w):
        return jnp.kron(eye4, w)

    def rowt(b):
        return jnp.tile(b.reshape(1, -1), (1, PK))

    wd = {}
    w1 = p_e['W1']
    wd['w1c'] = jnp.tile(p_e['W1'][48:64], (1, PK))
    wd['b1e'] = rowt(p_e['b1'])
    wd['w2'] = bd4(p_e['W2'])
    wd['b2'] = rowt(p_e['b2'])
    wd['w3'] = bd4(p_e['W3'])
    wd['b3'] = rowt(p_e['b3'])
    wd['wd1'] = bd4(p_e['W3'] @ d_e['W1'])
    wd['bd1'] = rowt(p_e['b3'] @ d_e['W1'] + d_e['b1'])
    wd['d2'] = bd4(d_e['W2'])
    wd['dd2'] = rowt(d_e['b2'])
    q = jnp.array([[-1.0, 1.0], [1.0, -1.0]], F32)
    wd['d3q'] = bd4(d_e['W3'] @ q)
    wd['dd3q'] = rowt(d_e['b3'] @ q)
    wv1 = p_v['W1']
    wd['wv1'] = wv1[0:32]
    wd['wv1c'] = wv1[32:48]
    wd['bv1'] = row(p_v['b1'])
    wd['wv2'] = p_v['W2']
    wd['bv2'] = row(p_v['b2'])
    wd['wv3d'] = jnp.concatenate([p_v['W3'], p_v['W3'] @ d_v['W1']], axis=1)
    wd['bvd'] = row(jnp.concatenate([p_v['b3'], p_v['b3'] @ d_v['W1'] + d_v['b1']]))
    wd['dv2'] = d_v['W2']
    wd['ddv2'] = row(d_v['b2'])
    wd['dv3'] = d_v['W3']
    wd['ddv3'] = row(d_v['b3'])
    wd['we1'] = bd4(enc_e['W1'])
    wd['be1'] = rowt(enc_e['b1'])
    wd['we2'] = bd4(enc_e['W2'])
    wd['be2'] = rowt(enc_e['b2'])
    wd['wfa0'] = bd4(enc_e['W3'] @ w1[0:16])
    wd['wfb0'] = jnp.concatenate([bd4(w1[16:32]), bd4(w1[32:48])], axis=0)
    wd['b1e0'] = rowt(p_e['b1'] + enc_e['b3'] @ w1[0:16])
    wd['wfa1'] = jnp.concatenate(
        [bd4(w1[0:16]), bd4(w1[16:32]), bd4(w1[32:48])], axis=0)
    wc1 = p_c['W1']
    cw = (wc1[0:16], wc1[16:32], wc1[32:48], row(p_c['b1']),
          p_c['W2'], row(p_c['b2']),
          jnp.concatenate([p_c['W3'], p_c['W3'] @ d_c['W1']], axis=1),
          row(jnp.concatenate([p_c['b3'], p_c['b3'] @ d_c['W1'] + d_c['b1']])),
          d_c['W2'], row(d_c['b2']), d_c['W3'], row(d_c['b3']))
    return wd, cw


def kernel(vertex, edge, context, incoming, params):
    wd, cw = _prep_weights(params)
    vertex_f = vertex.reshape(BV, 8)
    edge_f = edge.reshape(NP, 4 * PK)
    inc2d = incoming.reshape(BV, D)

    boff = (jnp.arange(B, dtype=jnp.int32) * V)[:, None, None]
    idx_s = (jnp.maximum(incoming - 1, 0) + boff).reshape(N)
    vrow = jnp.arange(V, dtype=jnp.int32)[None, :, None]
    idx_r = jnp.broadcast_to(vrow + boff, (B, V, D)).reshape(N)
    idx_m = (incoming > 0).astype(jnp.int32).reshape(N)
    mtab = jnp.concatenate(
        [jnp.zeros((1, 16), F32), jnp.ones((1, 16), F32)], axis=0)

    emb_v, emb_c = _enc_v(vertex_f, context, params['enc_v'], params['enc_c'])

    mk = None
    outs_v, outs_e, outs_c = [], [], []
    ein = edge_f
    for t in range(T):
        if mk is None:
            sv, rv, mk = _sc_gather((emb_v, emb_v, mtab), (idx_s, idx_r, idx_m))
            mk = mk.reshape(NP, 16 * PK)
        else:
            sv, rv = _sc_gather((emb_v, emb_v), (idx_s, idx_r))
        sv = sv.reshape(NP, 16 * PK)
        rv = rv.reshape(NP, 16 * PK)

        wd_t = dict(wd)
        if t == 0:
            wd_t['wfa'], wd_t['wfb'], wd_t['b1e'] = wd['wfa0'], wd['wfb0'], wd['b1e0']
        else:
            wd_t['wfa'] = wd['wfa1']
        e_new, o_e, v_new, o_v, ge, gv, ne = _mega(
            t == 0, ein, sv, rv, mk, inc2d, emb_v, emb_c, wd_t)
        emb_c, o_c = _ctx(ge, gv, ne, emb_c, cw)
        emb_v = v_new
        ein = e_new
        outs_v.append(o_v.reshape(B, V, 2))
        outs_e.append(o_e.reshape(B, V, D, 2))
        outs_c.append(o_c)

    return (jnp.stack(outs_v), jnp.stack(outs_e), jnp.stack(outs_c))


# speedup vs baseline: 2.0409x; 1.9394x over previous
import functools

import jax
import jax.numpy as jnp
from jax import lax
from jax.experimental import pallas as pl
from jax.experimental.pallas import tpu as pltpu
from jax.experimental.pallas import tpu_sc as plsc

B, V, D = 4, 10000, 16
N = B * V * D
BV = B * V
TROWS = B * (V + 1)
VE = EE = CE = 16
T = 2
F32 = jnp.float32

NW = 32
PER_W = N // NW
CH = 1000
NCH = PER_W // CH


def _sc_gather(tables, idxs):
    ns = len(tables)
    mesh = plsc.VectorSubcoreMesh(core_axis_name="c", subcore_axis_name="s")
    scratch = ([pltpu.VMEM((PER_W,), jnp.int32)] * ns
               + [pltpu.VMEM((CH, 16), F32)] * (2 * ns)
               + [pltpu.SemaphoreType.DMA] * (2 * ns))

    @functools.partial(
        pl.kernel,
        mesh=mesh,
        compiler_params=pltpu.CompilerParams(use_tc_tiling_on_sc=False),
        out_type=tuple(jax.ShapeDtypeStruct((N, 16), F32) for _ in range(ns)),
        scratch_types=scratch,
    )
    def gk(*refs):
        t_hbm = refs[0:ns]
        i_hbm = refs[ns:2 * ns]
        o_hbm = refs[2 * ns:3 * ns]
        k = 3 * ns
        i_v = refs[k:k + ns]
        bufs = [(refs[k + ns + 2 * j], refs[k + ns + 2 * j + 1]) for j in range(ns)]
        sems = [(refs[k + 3 * ns + 2 * j], refs[k + 3 * ns + 2 * j + 1])
                for j in range(ns)]
        wid = lax.axis_index("s") * 2 + lax.axis_index("c")
        base = wid * PER_W

        for j in range(ns):
            pltpu.sync_copy(i_hbm[j].at[pl.ds(base, PER_W)], i_v[j])

        def dsc(j, c, s):
            return pltpu.make_async_copy(
                t_hbm[j].at[i_v[j].at[pl.ds(c * CH, CH)]], bufs[j][s], sems[j][s])

        def start(c, s):
            for j in range(ns):
                dsc(j, c, s).start()

        def wait(c, s):
            for j in range(ns):
                dsc(j, c, s).wait()

        for s in range(2):
            start(s, s)

        def body(i, carry):
            for s in range(2):
                c = 2 * i + s
                wait(c, s)
                for j in range(ns):
                    pltpu.sync_copy(bufs[j][s],
                                    o_hbm[j].at[pl.ds(base + c * CH, CH)])

                @pl.when(c + 2 < NCH)
                def _():
                    start(c + 2, s)
            return carry

        lax.fori_loop(0, NCH // 2, body, 0)

    return gk(*tables, *idxs)


VB = 400
EB = VB * D
GRID = BV // VB
GPB = V // VB
PK = 4
EBP = EB // PK
NP = N // PK


def _relu(x):
    return jnp.maximum(x, 0.0)


def _full(w):
    return pl.BlockSpec(w.shape, lambda *g: tuple(0 for _ in w.shape))


_ENC_VB = 2000


def _enc_v_body(vx, ctx, inc, wv1, bv1, wv2, bv2, wv3, bv3,
                wc1, bc1, wc2, bc2, wc3, bc3, out_v, out_c, o_is, o_ir):
    g = pl.program_id(0)
    h = _relu(vx[...] @ wv1[...] + bv1[...])
    h = _relu(h @ wv2[...] + bv2[...])
    out_v[...] = h @ wv3[...] + bv3[...]
    bofs = (g // (V // _ENC_VB)) * V
    o_is[...] = jnp.maximum(inc[...] - 1, 0) + bofs
    o_ir[...] = (jax.lax.broadcasted_iota(jnp.int32, (_ENC_VB, D), 0)
                 + g * _ENC_VB)

    @pl.when(g == 0)
    def _():
        hc = _relu(ctx[...] @ wc1[...] + bc1[...])
        hc = _relu(hc @ wc2[...] + bc2[...])
        out_c[...] = hc @ wc3[...] + bc3[...]


def _enc_v(vertex_f, context, inc2d, pv, pc):
    vb = _ENC_VB
    ws = (pv['W1'], pv['b1'], pv['W2'], pv['b2'], pv['W3'], pv['b3'],
          pc['W1'], pc['b1'], pc['W2'], pc['b2'], pc['W3'], pc['b3'])
    return pl.pallas_call(
        _enc_v_body,
        grid=(BV // vb,),
        in_specs=[pl.BlockSpec((vb, 8), lambda g: (g, 0)),
                  _full(context),
                  pl.BlockSpec((vb, D), lambda g: (g, 0))]
        + [_full(w) for w in ws],
        out_specs=[pl.BlockSpec((vb, VE), lambda g: (g, 0)),
                   pl.BlockSpec((B, CE), lambda g: (0, 0)),
                   pl.BlockSpec((vb, D), lambda g: (g, 0)),
                   pl.BlockSpec((vb, D), lambda g: (g, 0))],
        out_shape=[jax.ShapeDtypeStruct((BV, VE), F32),
                   jax.ShapeDtypeStruct((B, CE), F32),
                   jax.ShapeDtypeStruct((BV, D), jnp.int32),
                   jax.ShapeDtypeStruct((BV, D), jnp.int32)],
    )(vertex_f, context, inc2d, *ws)


def _mega_tail(mask16, inc, embv, embc, h, wv, o_e, o_oe, o_v, o_ov, o_ge, o_gv, o_ne):
    g = pl.program_id(0)
    b = g // GPB
    h = _relu(h @ wv['w2'][...] + wv['b2'][...])
    e_new = h @ wv['w3'][...] + wv['b3'][...]
    o_e[...] = e_new
    dh = _relu(h @ wv['wd1'][...] + wv['bd1'][...])
    dh = _relu(dh @ wv['d2'][...] + wv['dd2'][...])
    w = dh @ wv['d3q'][...] + wv['dd3q'][...]
    sp = jnp.maximum(w, 0.0) + jnp.log(1.0 + jnp.exp(-jnp.abs(w)))
    mask2 = jnp.concatenate([mask16[:, 0:2], mask16[:, 16:18],
                             mask16[:, 32:34], mask16[:, 48:50]], axis=1)
    o_oe[...] = mask2 * (-sp)

    m_e = mask16 * e_new
    es4 = jnp.sum(m_e.reshape(VB, PK, 64), axis=1)
    es = (es4[:, 0:16] + es4[:, 16:32] + es4[:, 32:48] + es4[:, 48:64])
    ns = jnp.sum((inc[...] > 0).astype(F32), axis=1, keepdims=True)
    avg_e = es / jnp.maximum(ns, 1.0)
    xv = jnp.concatenate([avg_e, embv[...]], axis=1)
    ccv = embc[pl.ds(b, 1), :] @ wv['wv1c'][...]
    hv = _relu(xv @ wv['wv1'][...] + ccv + wv['bv1'][...])
    hv = _relu(hv @ wv['wv2'][...] + wv['bv2'][...])
    ov80 = hv @ wv['wv3d'][...] + wv['bvd'][...]
    v_new = ov80[:, 0:16]
    o_v[...] = v_new
    dv = _relu(ov80[:, 16:80])
    dv = _relu(dv @ wv['dv2'][...] + wv['ddv2'][...])
    o_ov[...] = dv @ wv['dv3'][...] + wv['ddv3'][...]

    @pl.when(g == 0)
    def _():
        o_ge[...] = jnp.zeros_like(o_ge)
        o_gv[...] = jnp.zeros_like(o_gv)
        o_ne[...] = jnp.zeros_like(o_ne)

    ones1 = jnp.ones((1, VB), F32)
    o_ge[pl.ds(b, 1), :] += ones1 @ es
    o_gv[pl.ds(b, 1), :] += ones1 @ v_new
    o_ne[pl.ds(b, 1), :] += jnp.broadcast_to(ones1 @ ns, (1, 16))


_TAIL_KEYS = ('w2', 'b2', 'w3', 'b3', 'wd1', 'bd1', 'd2', 'dd2', 'd3q', 'dd3q',
              'wv1', 'wv1c', 'bv1', 'wv2', 'bv2', 'wv3d', 'bvd',
              'dv2', 'ddv2', 'dv3', 'ddv3')
_T0_KEYS = ('we1', 'be1', 'we2', 'be2', 'wfa', 'wfr', 'wfs', 'w1c', 'b1e') + _TAIL_KEYS
_T1_KEYS = ('wfe', 'wfr', 'wfs', 'w1c', 'b1e') + _TAIL_KEYS


def _mega_body_t0(ein, sv, rv, mk, inc, embv, embc, *rest):
    nw = len(_T0_KEYS)
    wv = dict(zip(_T0_KEYS, rest[:nw]))
    outs = rest[nw:]
    mask16 = mk[...]
    he = _relu(ein[...] @ wv['we1'][...] + wv['be1'][...])
    he = _relu(he @ wv['we2'][...] + wv['be2'][...])
    cc = embc[pl.ds(pl.program_id(0) // GPB, 1), :] @ wv['w1c'][...]
    h = _relu(he @ wv['wfa'][...] + (mask16 * rv[...]) @ wv['wfr'][...]
              + (mask16 * sv[...]) @ wv['wfs'][...] + cc + wv['b1e'][...])
    _mega_tail(mask16, inc, embv, embc, h, wv, *outs)


def _mega_body_t1(ein, sv, rv, mk, inc, embv, embc, *rest):
    nw = len(_T1_KEYS)
    wv = dict(zip(_T1_KEYS, rest[:nw]))
    outs = rest[nw:]
    mask16 = mk[...]
    cc = embc[pl.ds(pl.program_id(0) // GPB, 1), :] @ wv['w1c'][...]
    h = _relu(ein[...] @ wv['wfe'][...] + (mask16 * rv[...]) @ wv['wfr'][...]
              + (mask16 * sv[...]) @ wv['wfs'][...] + cc + wv['b1e'][...])
    _mega_tail(mask16, inc, embv, embc, h, wv, *outs)


def _mega(t0, ein, sv, rv, mk, inc2d, embv, embc, wd):
    keys = _T0_KEYS if t0 else _T1_KEYS
    ein_w = ein.shape[1]
    body = _mega_body_t0 if t0 else _mega_body_t1
    return pl.pallas_call(
        body,
        grid=(GRID,),
        in_specs=[
            pl.BlockSpec((EBP, ein_w), lambda g: (g, 0)),
            pl.BlockSpec((EBP, 64), lambda g: (g, 0)),
            pl.BlockSpec((EBP, 64), lambda g: (g, 0)),
            pl.BlockSpec((EBP, 64), lambda g: (g, 0)),
            pl.BlockSpec((VB, D), lambda g: (g, 0)),
            pl.BlockSpec((VB, VE), lambda g: (g, 0)),
            pl.BlockSpec((B, CE), lambda g: (0, 0)),
        ] + [_full(wd[k]) for k in keys],
        out_specs=[
            pl.BlockSpec((EBP, 64), lambda g: (g, 0)),
            pl.BlockSpec((EBP, 8), lambda g: (g, 0)),
            pl.BlockSpec((VB, VE), lambda g: (g, 0)),
            pl.BlockSpec((VB, 2), lambda g: (g, 0)),
            pl.BlockSpec((B, EE), lambda g: (0, 0)),
            pl.BlockSpec((B, VE), lambda g: (0, 0)),
            pl.BlockSpec((B, 16), lambda g: (0, 0)),
        ],
        out_shape=[
            jax.ShapeDtypeStruct((NP, 64), F32),
            jax.ShapeDtypeStruct((NP, 8), F32),
            jax.ShapeDtypeStruct((BV, VE), F32),
            jax.ShapeDtypeStruct((BV, 2), F32),
            jax.ShapeDtypeStruct((B, EE), F32),
            jax.ShapeDtypeStruct((B, VE), F32),
            jax.ShapeDtypeStruct((B, 16), F32),
        ],
    )(ein, sv, rv, mk, inc2d, embv, embc, *[wd[k] for k in keys])


def _ctx_body(ge, gv, ne, embc, w1e, w1v, w1c, b1, w2, b2, w3d, bd,
              dc2, ddc2, dc3, ddc3, o_c, o_oc):
    gee = ge[...] / ne[:, 0:1]
    gvv = gv[...] * (1.0 / V)
    h = _relu(gee @ w1e[...] + gvv @ w1v[...] + embc[...] @ w1c[...] + b1[...])
    h = _relu(h @ w2[...] + b2[...])
    o80 = h @ w3d[...] + bd[...]
    o_c[...] = o80[:, 0:16]
    dh = _relu(o80[:, 16:80])
    dh = _relu(dh @ dc2[...] + ddc2[...])
    o_oc[...] = dh @ dc3[...] + ddc3[...]


def _ctx(ge, gv, ne, embc, cw):
    args = (ge, gv, ne, embc) + cw
    return pl.pallas_call(
        _ctx_body,
        in_specs=[_full(a) for a in args],
        out_specs=[pl.BlockSpec((B, CE), lambda *g: (0, 0)),
                   pl.BlockSpec((B, 2), lambda *g: (0, 0))],
        out_shape=[jax.ShapeDtypeStruct((B, CE), F32),
                   jax.ShapeDtypeStruct((B, 2), F32)],
    )(*args)


def _prep_weights(params):
    p_e, p_v, p_c = params['e_net'], params['v_net'], params['c_net']
    d_e, d_v, d_c = params['dec_e'], params['dec_v'], params['dec_c']
    enc_e = params['enc_e']

    def row(b):
        return b.reshape(1, -1)

    eye4 = jnp.eye(PK, dtype=F32)

    def bd4(w):
        return jnp.kron(eye4, w)

    def rowt(b):
        return jnp.tile(b.reshape(1, -1), (1, PK))

    wd = {}
    w1 = p_e['W1']
    wd['w1c'] = jnp.tile(p_e['W1'][48:64], (1, PK))
    wd['b1e'] = rowt(p_e['b1'])
    wd['w2'] = bd4(p_e['W2'])
    wd['b2'] = rowt(p_e['b2'])
    wd['w3'] = bd4(p_e['W3'])
    wd['b3'] = rowt(p_e['b3'])
    wd['wd1'] = bd4(p_e['W3'] @ d_e['W1'])
    wd['bd1'] = rowt(p_e['b3'] @ d_e['W1'] + d_e['b1'])
    wd['d2'] = bd4(d_e['W2'])
    wd['dd2'] = rowt(d_e['b2'])
    q = jnp.array([[-1.0, 1.0], [1.0, -1.0]], F32)
    wd['d3q'] = bd4(d_e['W3'] @ q)
    wd['dd3q'] = rowt(d_e['b3'] @ q)
    wv1 = p_v['W1']
    wd['wv1'] = wv1[0:32]
    wd['wv1c'] = wv1[32:48]
    wd['bv1'] = row(p_v['b1'])
    wd['wv2'] = p_v['W2']
    wd['bv2'] = row(p_v['b2'])
    wd['wv3d'] = jnp.concatenate([p_v['W3'], p_v['W3'] @ d_v['W1']], axis=1)
    wd['bvd'] = row(jnp.concatenate([p_v['b3'], p_v['b3'] @ d_v['W1'] + d_v['b1']]))
    wd['dv2'] = d_v['W2']
    wd['ddv2'] = row(d_v['b2'])
    wd['dv3'] = d_v['W3']
    wd['ddv3'] = row(d_v['b3'])
    wd['we1'] = bd4(enc_e['W1'])
    wd['be1'] = rowt(enc_e['b1'])
    wd['we2'] = bd4(enc_e['W2'])
    wd['be2'] = rowt(enc_e['b2'])
    wd['wfa0'] = bd4(enc_e['W3'] @ w1[0:16])
    wd['b1e0'] = rowt(p_e['b1'] + enc_e['b3'] @ w1[0:16])
    wd['wfe'] = bd4(w1[0:16])
    wd['wfr'] = bd4(w1[16:32])
    wd['wfs'] = bd4(w1[32:48])
    wc1 = p_c['W1']
    cw = (wc1[0:16], wc1[16:32], wc1[32:48], row(p_c['b1']),
          p_c['W2'], row(p_c['b2']),
          jnp.concatenate([p_c['W3'], p_c['W3'] @ d_c['W1']], axis=1),
          row(jnp.concatenate([p_c['b3'], p_c['b3'] @ d_c['W1'] + d_c['b1']])),
          d_c['W2'], row(d_c['b2']), d_c['W3'], row(d_c['b3']))
    return wd, cw


def kernel(vertex, edge, context, incoming, params):
    wd, cw = _prep_weights(params)
    vertex_f = vertex.reshape(BV, 8)
    edge_f = edge.reshape(NP, 4 * PK)
    inc2d = incoming.reshape(BV, D)

    idx_m = incoming.reshape(N)
    mtab = jnp.concatenate(
        [jnp.zeros((1, 16), F32), jnp.ones((V, 16), F32)], axis=0)

    (mk,) = _sc_gather((mtab,), (idx_m,))
    mk = mk.reshape(NP, 16 * PK)
    emb_v, emb_c, idx_s2, idx_r2 = _enc_v(
        vertex_f, context, inc2d, params['enc_v'], params['enc_c'])
    idx_s = idx_s2.reshape(N)
    idx_r = idx_r2.reshape(N)

    outs_v, outs_e, outs_c = [], [], []
    ein = edge_f
    for t in range(T):
        sv, rv = _sc_gather((emb_v, emb_v), (idx_s, idx_r))
        sv = sv.reshape(NP, 16 * PK)
        rv = rv.reshape(NP, 16 * PK)

        wd_t = dict(wd)
        if t == 0:
            wd_t['wfa'], wd_t['b1e'] = wd['wfa0'], wd['b1e0']
        e_new, o_e, v_new, o_v, ge, gv, ne = _mega(
            t == 0, ein, sv, rv, mk, inc2d, emb_v, emb_c, wd_t)
        emb_c, o_c = _ctx(ge, gv, ne, emb_c, cw)
        emb_v = v_new
        ein = e_new
        outs_v.append(o_v.reshape(B, V, 2))
        outs_e.append(o_e.reshape(B, V, D, 2))
        outs_c.append(o_c)

    return (jnp.stack(outs_v), jnp.stack(outs_e), jnp.stack(outs_c))
